# baseline (speedup 1.0000x reference)
import functools

import jax
import jax.numpy as jnp
from jax import lax
from jax.experimental import pallas as pl
from jax.experimental.pallas import tpu as pltpu
from jax.experimental.pallas import tpu_sc as plsc

N_INT = 26
N_DISC = 2
N_FEAT = N_INT + N_DISC
EMB = 32
BATCH = 16384
N_BINS = 100
VOCAB = 100000

NC = 2
NS = 16
NW = NC * NS
BPW = BATCH // NW
LANES = 16

KA = 16
KB_INT = N_INT - KA
KB = KB_INT + N_DISC

CHUNK = 128
NCHUNK = BPW // CHUNK
NBUF_A = 4
NBUF_B = 3

NROWS_B = [VOCAB + 1] * KB_INT + [N_BINS + 1] * N_DISC

_mesh = plsc.VectorSubcoreMesh(core_axis_name="c", subcore_axis_name="s")
_params = pltpu.CompilerParams(
    needs_layout_passes=False, use_tc_tiling_on_sc=False)


@functools.partial(
    pl.kernel,
    out_type=jax.ShapeDtypeStruct((BATCH, KA * EMB), jnp.float32),
    mesh=_mesh,
    compiler_params=_params,
    scratch_types=[
        pltpu.VMEM((KA, NCHUNK, CHUNK), jnp.int32),
        pltpu.VMEM((KA, BPW), jnp.int32),
        pltpu.VMEM((NBUF_A, BPW, EMB), jnp.float32),
        pltpu.SemaphoreType.DMA,
        pltpu.SemaphoreType.DMA,
        pltpu.SemaphoreType.DMA,
    ],
)
def _sc_rows(*refs):
    idx_hbm = refs[:KA]
    tables = refs[KA:2 * KA]
    out = refs[2 * KA]
    idx_v, raw_v, rows_v, lsem, gsem, osem = refs[-6:]

    wid = lax.axis_index("c") * NS + lax.axis_index("s")
    base = wid * BPW

    loads = []
    for f in range(KA):
        loads.append(pltpu.make_async_copy(
            idx_hbm[f].at[pl.ds(base, BPW)], raw_v.at[f], lsem))
    for cp in loads:
        cp.start()
    for cp in loads:
        cp.wait()

    for f in range(KA):
        for j in range(NCHUNK):
            def int_body(i, _, f=f, j=j):
                sl = pl.ds(i * LANES, LANES)
                idx_v[f, j, sl] = (
                    raw_v[f, pl.ds(j * CHUNK + i * LANES, LANES)] + 1)
                return 0
            lax.fori_loop(0, CHUNK // LANES, int_body, 0)

    gcps = [None] * NBUF_A
    ocps = [None] * NBUF_A

    def fire(f):
        b = f % NBUF_A
        cps = []
        for j in range(NCHUNK):
            cps.append(pltpu.make_async_copy(
                tables[f].at[idx_v.at[f, j]],
                rows_v.at[b, pl.ds(j * CHUNK, CHUNK)],
                gsem))
        for cp in cps:
            cp.start()
        gcps[b] = cps

    fire(0)
    if KA > 1:
        fire(1)
    for f in range(KA):
        b = f % NBUF_A
        for cp in gcps[b]:
            cp.wait()
        ocps[b] = pltpu.make_async_copy(
            rows_v.at[b], out.at[pl.ds(base, BPW), pl.ds(f * EMB, EMB)], osem)
        ocps[b].start()
        nf = f + 2
        if nf < KA:
            nb = nf % NBUF_A
            if ocps[nb] is not None:
                ocps[nb].wait()
                ocps[nb] = None
            fire(nf)
    for b in range(NBUF_A):
        if ocps[b] is not None:
            ocps[b].wait()


@functools.partial(
    pl.kernel,
    out_type=jax.ShapeDtypeStruct((KB, EMB, BATCH), jnp.float32),
    mesh=_mesh,
    compiler_params=_params,
    scratch_types=[
        pltpu.VMEM((KB, BPW), jnp.int32),
        pltpu.VMEM((N_DISC, BPW), jnp.float32),
        pltpu.VMEM((128,), jnp.float32),
        pltpu.VMEM((128,), jnp.float32),
        pltpu.VMEM((NBUF_B, EMB, BPW), jnp.int32),
        pltpu.VMEM((NBUF_B, EMB, BPW), jnp.float32),
        pltpu.SemaphoreType.DMA,
        pltpu.SemaphoreType.DMA,
        pltpu.SemaphoreType.DMA,
    ],
)
def _sc_elems(*refs):
    idx_hbm = refs[:KB_INT]
    disc_hbm = refs[KB_INT:KB_INT + N_DISC]
    bins_hbm = refs[KB_INT + N_DISC:KB_INT + 2 * N_DISC]
    tables = refs[KB_INT + 2 * N_DISC:KB_INT + 2 * N_DISC + KB]
    out = refs[KB_INT + 2 * N_DISC + KB]
    adj_v, disc_v, bins0_v, bins1_v, eidx_v, rows_v, lsem, gsem, osem = refs[-9:]
    bins_v = (bins0_v, bins1_v)

    wid = lax.axis_index("c") * NS + lax.axis_index("s")
    base = wid * BPW

    loads = []
    for f in range(KB_INT):
        loads.append(pltpu.make_async_copy(
            idx_hbm[f].at[pl.ds(base, BPW)], adj_v.at[f], lsem))
    for d in range(N_DISC):
        loads.append(pltpu.make_async_copy(
            disc_hbm[d].at[pl.ds(base, BPW)], disc_v.at[d], lsem))
        loads.append(pltpu.make_async_copy(
            bins_hbm[d], bins_v[d].at[pl.ds(0, N_BINS)], lsem))
    for cp in loads:
        cp.start()
    for cp in loads:
        cp.wait()

    for f in range(KB_INT):
        def int_body(i, _, f=f):
            sl = pl.ds(i * LANES, LANES)
            adj_v[f, sl] = adj_v[f, sl] + 1
            return 0
        lax.fori_loop(0, BPW // LANES, int_body, 0)
    for d in range(N_DISC):
        def disc_body(i, _, d=d):
            sl = pl.ds(i * LANES, LANES)
            x = disc_v[d, sl]
            pos = jnp.zeros((LANES,), jnp.int32)
            for s in (64, 32, 16, 8, 4, 2, 1):
                probe = pos + (s - 1)
                bv = plsc.load_gather(
                    bins_v[d], [jnp.minimum(probe, N_BINS - 1)])
                take = (bv <= x) & (probe <= N_BINS - 1)
                pos = jnp.where(take, pos + s, pos)
            adj_v[KB_INT + d, sl] = pos
            return 0
        lax.fori_loop(0, BPW // LANES, disc_body, 0)

    gcps = [None] * NBUF_B
    ocps = [None] * NBUF_B

    def expand(f):
        b = f % NBUF_B
        nr = NROWS_B[f]
        def body(i, _, f=f, b=b, nr=nr):
            sl = pl.ds(i * LANES, LANES)
            a = adj_v[f, sl]
            for e in range(EMB):
                eidx_v[b, e, sl] = a + (e * nr)
            return 0
        lax.fori_loop(0, BPW // LANES, body, 0)

    def fire(f):
        b = f % NBUF_B
        if ocps[b] is not None:
            ocps[b].wait()
            ocps[b] = None
        cps = []
        for e in range(EMB):
            cps.append(pltpu.make_async_copy(
                tables[f].at[eidx_v.at[b, e]], rows_v.at[b, e], gsem))
        for cp in cps:
            cp.start()
        gcps[b] = cps

    expand(0)
    fire(0)
    if KB > 1:
        expand(1)
    for f in range(KB):
        b = f % NBUF_B
        if f + 1 < KB:
            fire(f + 1)
        for cp in gcps[b]:
            cp.wait()
        ocps[b] = pltpu.make_async_copy(
            rows_v.at[b], out.at[f, :, pl.ds(base, BPW)], osem)
        ocps[b].start()
        if f + 2 < KB:
            expand(f + 2)
    for b in range(NBUF_B):
        if ocps[b] is not None:
            ocps[b].wait()


def kernel(int_0, table_int_0, int_1, table_int_1, int_2, table_int_2,
           int_3, table_int_3, int_4, table_int_4, int_5, table_int_5,
           int_6, table_int_6, int_7, table_int_7, int_8, table_int_8,
           int_9, table_int_9, int_10, table_int_10, int_11, table_int_11,
           int_12, table_int_12, int_13, table_int_13, int_14, table_int_14,
           int_15, table_int_15, int_16, table_int_16, int_17, table_int_17,
           int_18, table_int_18, int_19, table_int_19, int_20, table_int_20,
           int_21, table_int_21, int_22, table_int_22, int_23, table_int_23,
           int_24, table_int_24, int_25, table_int_25,
           disc_0, table_disc_0, bins_0, disc_1, table_disc_1, bins_1):
    kw = dict(locals())
    ints = [kw['int_%d' % i] for i in range(N_INT)]
    itabs = [kw['table_int_%d' % i] for i in range(N_INT)]
    discs = [kw['disc_%d' % i] for i in range(N_DISC)]
    bins = [kw['bins_%d' % i] for i in range(N_DISC)]
    dtabs = [kw['table_disc_%d' % i] for i in range(N_DISC)]

    flat_b = ([t.T.reshape(-1) for t in itabs[KA:]]
              + [t.T.reshape(-1) for t in dtabs])
    out_b = _sc_elems(*ints[KA:], *discs, *bins, *flat_b)
    out_a = _sc_rows(*ints[:KA], *itabs[:KA])
    out_b_t = jnp.transpose(out_b, (2, 0, 1)).reshape(BATCH, KB * EMB)
    return jnp.concatenate([out_a, out_b_t], axis=1)


# speedup vs baseline: 1.1299x; 1.1299x over previous
import functools

import jax
import jax.numpy as jnp
from jax import lax
from jax.experimental import pallas as pl
from jax.experimental.pallas import tpu as pltpu
from jax.experimental.pallas import tpu_sc as plsc

N_INT = 26
N_DISC = 2
N_FEAT = N_INT + N_DISC
EMB = 32
BATCH = 16384
N_BINS = 100

NC = 2
NS = 16
NW = NC * NS
BPW = BATCH // NW
CHUNK = 128
NCHUNK = BPW // CHUNK
NBUF = 4
LANES = 16

_mesh = plsc.VectorSubcoreMesh(core_axis_name="c", subcore_axis_name="s")


@functools.partial(
    pl.kernel,
    out_type=jax.ShapeDtypeStruct((BATCH, N_FEAT * EMB), jnp.float32),
    mesh=_mesh,
    compiler_params=pltpu.CompilerParams(
        needs_layout_passes=False, use_tc_tiling_on_sc=False),
    scratch_types=[
        pltpu.VMEM((N_FEAT, NCHUNK, CHUNK), jnp.int32),
        pltpu.VMEM((N_INT, BPW), jnp.int32),
        pltpu.VMEM((N_DISC, BPW), jnp.float32),
        pltpu.VMEM((128,), jnp.float32),
        pltpu.VMEM((128,), jnp.float32),
        pltpu.VMEM((NBUF, BPW, EMB), jnp.float32),
        pltpu.SemaphoreType.DMA,
        pltpu.SemaphoreType.DMA,
        pltpu.SemaphoreType.DMA,
    ],
)
def _sc_lookup(*refs):
    idx_hbm = refs[:N_INT]
    disc_hbm = refs[N_INT:N_INT + N_DISC]
    bins_hbm = refs[N_INT + N_DISC:N_INT + 2 * N_DISC]
    tables = refs[N_INT + 2 * N_DISC:N_INT + 2 * N_DISC + N_FEAT]
    out = refs[N_INT + 2 * N_DISC + N_FEAT]
    idx_v, raw_v, disc_v, bins0_v, bins1_v, rows_v, lsem, gsem, osem = refs[-9:]
    bins_v = (bins0_v, bins1_v)

    wid = lax.axis_index("c") * NS + lax.axis_index("s")
    base = wid * BPW

    loads = []
    for f in range(N_INT):
        loads.append(pltpu.make_async_copy(
            idx_hbm[f].at[pl.ds(base, BPW)], raw_v.at[f], lsem))
    for d in range(N_DISC):
        loads.append(pltpu.make_async_copy(
            disc_hbm[d].at[pl.ds(base, BPW)], disc_v.at[d], lsem))
        loads.append(pltpu.make_async_copy(
            bins_hbm[d], bins_v[d].at[pl.ds(0, N_BINS)], lsem))
    for cp in loads:
        cp.start()
    for cp in loads:
        cp.wait()

    for f in range(N_INT):
        for j in range(NCHUNK):
            def int_body(i, _, f=f, j=j):
                sl = pl.ds(i * LANES, LANES)
                idx_v[f, j, sl] = raw_v[f, pl.ds(j * CHUNK + i * LANES, LANES)] + 1
                return 0
            lax.fori_loop(0, CHUNK // LANES, int_body, 0)
    for d in range(N_DISC):
        for j in range(NCHUNK):
            def disc_body(i, _, d=d, j=j):
                x = disc_v[d, pl.ds(j * CHUNK + i * LANES, LANES)]
                pos = jnp.zeros((LANES,), jnp.int32)
                for s in (64, 32, 16, 8, 4, 2, 1):
                    probe = pos + (s - 1)
                    bv = plsc.load_gather(
                        bins_v[d], [jnp.minimum(probe, N_BINS - 1)])
                    take = (bv <= x) & (probe <= N_BINS - 1)
                    pos = jnp.where(take, pos + s, pos)
                idx_v[N_INT + d, j, pl.ds(i * LANES, LANES)] = pos
                return 0
            lax.fori_loop(0, CHUNK // LANES, disc_body, 0)

    gcps = [None] * NBUF
    ocps = [None] * NBUF

    def fire(f):
        b = f % NBUF
        cps = []
        for j in range(NCHUNK):
            cps.append(pltpu.make_async_copy(
                tables[f].at[idx_v.at[f, j]],
                rows_v.at[b, pl.ds(j * CHUNK, CHUNK)],
                gsem))
        for cp in cps:
            cp.start()
        gcps[b] = cps

    fire(0)
    if N_FEAT > 1:
        fire(1)
    for f in range(N_FEAT):
        b = f % NBUF
        for cp in gcps[b]:
            cp.wait()
        ocps[b] = pltpu.make_async_copy(
            rows_v.at[b], out.at[pl.ds(base, BPW), pl.ds(f * EMB, EMB)], osem)
        ocps[b].start()
        nf = f + 2
        if nf < N_FEAT:
            nb = nf % NBUF
            if ocps[nb] is not None:
                ocps[nb].wait()
                ocps[nb] = None
            fire(nf)
    for b in range(NBUF):
        if ocps[b] is not None:
            ocps[b].wait()


def kernel(int_0, table_int_0, int_1, table_int_1, int_2, table_int_2,
           int_3, table_int_3, int_4, table_int_4, int_5, table_int_5,
           int_6, table_int_6, int_7, table_int_7, int_8, table_int_8,
           int_9, table_int_9, int_10, table_int_10, int_11, table_int_11,
           int_12, table_int_12, int_13, table_int_13, int_14, table_int_14,
           int_15, table_int_15, int_16, table_int_16, int_17, table_int_17,
           int_18, table_int_18, int_19, table_int_19, int_20, table_int_20,
           int_21, table_int_21, int_22, table_int_22, int_23, table_int_23,
           int_24, table_int_24, int_25, table_int_25,
           disc_0, table_disc_0, bins_0, disc_1, table_disc_1, bins_1):
    kw = dict(locals())
    ints = [kw['int_%d' % i] for i in range(N_INT)]
    discs = [kw['disc_%d' % i] for i in range(N_DISC)]
    bins = [kw['bins_%d' % i] for i in range(N_DISC)]
    tabs = ([kw['table_int_%d' % i] for i in range(N_INT)]
            + [kw['table_disc_%d' % i] for i in range(N_DISC)])
    return _sc_lookup(*ints, *discs, *bins, *tabs)
